# baseline (speedup 1.0000x reference)
import functools

import jax
import jax.numpy as jnp
from jax import lax
from jax.experimental import pallas as pl
from jax.experimental.pallas import tpu as pltpu
from jax.experimental.pallas import tpu_sc as plsc

DIM = 128
_NC, _NS = 2, 16
_NW = _NC * _NS
_CHUNK = 128
_NBUF = 4


@functools.lru_cache(maxsize=None)
def _build_sc_gather(n_rows: int):
    assert n_rows % (8 * _NW) == 0
    b_per_w = n_rows // _NW
    n_chunks = b_per_w // _CHUNK
    mesh = plsc.VectorSubcoreMesh(core_axis_name="c", subcore_axis_name="s")

    @functools.partial(
        pl.kernel,
        mesh=mesh,
        out_type=[
            jax.ShapeDtypeStruct((n_rows, DIM), jnp.float32),
            jax.ShapeDtypeStruct((n_rows, DIM), jnp.float32),
        ],
        scratch_types=[
            pltpu.VMEM((b_per_w,), jnp.int32),
            pltpu.VMEM((_NBUF, _CHUNK, DIM), jnp.float32),
            [pltpu.SemaphoreType.DMA] * _NBUF,
            [pltpu.SemaphoreType.DMA] * _NBUF,
        ],
    )
    def sc_gather(pos_hbm, cos_hbm, sin_hbm, cos_out, sin_out,
                  idx_v, bufs, sems_g, sems_w):
        wid = lax.axis_index("s") * _NC + lax.axis_index("c")
        base = wid * b_per_w
        pltpu.sync_copy(pos_hbm.at[pl.ds(base, b_per_w)], idx_v)
        chunks = [(t, o, j) for j in range(n_chunks)
                  for (t, o) in ((cos_hbm, cos_out), (sin_hbm, sin_out))]
        depth = _NBUF - 1
        n = len(chunks)
        g_cps = [None] * _NBUF
        w_cps = [None] * _NBUF
        for c in range(n + depth):
            if c < n:
                b = c % _NBUF
                table, _, j = chunks[c]
                if w_cps[b] is not None:
                    w_cps[b].wait()
                g_cps[b] = pltpu.async_copy(
                    table.at[idx_v.at[pl.ds(j * _CHUNK, _CHUNK)]],
                    bufs.at[b], sems_g[b])
            d = c - depth
            if 0 <= d < n:
                db = d % _NBUF
                _, out, j = chunks[d]
                g_cps[db].wait()
                w_cps[db] = pltpu.async_copy(
                    bufs.at[db], out.at[pl.ds(base + j * _CHUNK, _CHUNK)],
                    sems_w[db])
        for b in range(_NBUF):
            if w_cps[b] is not None:
                w_cps[b].wait()

    return sc_gather


def kernel(x, position_ids, cos_cached, sin_cached):
    b, s = position_ids.shape
    pos = position_ids.reshape(-1).astype(jnp.int32)
    cos_flat, sin_flat = _build_sc_gather(b * s)(
        pos, cos_cached.astype(jnp.float32), sin_cached.astype(jnp.float32))
    return (cos_flat.reshape(b, 1, s, DIM).astype(x.dtype),
            sin_flat.reshape(b, 1, s, DIM).astype(x.dtype))


# speedup vs baseline: 1.0087x; 1.0087x over previous
import functools

import jax
import jax.numpy as jnp
from jax import lax
from jax.experimental import pallas as pl
from jax.experimental.pallas import tpu as pltpu
from jax.experimental.pallas import tpu_sc as plsc

DIM = 128
_NC, _NS = 2, 16
_NW = _NC * _NS


@functools.lru_cache(maxsize=None)
def _build_sc_gather(b: int, s: int):
    n_rows = b * s
    assert n_rows % (8 * _NW) == 0
    b_per_w = n_rows // _NW
    w_per_row = s // b_per_w
    mesh = plsc.VectorSubcoreMesh(core_axis_name="c", subcore_axis_name="s")

    @functools.partial(
        pl.kernel,
        mesh=mesh,
        out_type=[
            jax.ShapeDtypeStruct((b, 1, s, DIM), jnp.float32),
            jax.ShapeDtypeStruct((b, 1, s, DIM), jnp.float32),
        ],
        scratch_types=[
            pltpu.VMEM((b_per_w,), jnp.int32),
            pltpu.VMEM((b_per_w, DIM), jnp.float32),
            pltpu.SemaphoreType.DMA,
        ],
    )
    def sc_gather(pos_hbm, cos_hbm, sin_hbm, cos_out, sin_out,
                  idx_v, rows_v, sem):
        wid = lax.axis_index("s") * _NC + lax.axis_index("c")
        r = wid // w_per_row
        c0 = (wid % w_per_row) * b_per_w
        pltpu.sync_copy(pos_hbm.at[r, pl.ds(c0, b_per_w)], idx_v)
        for table, out in ((cos_hbm, cos_out), (sin_hbm, sin_out)):
            pltpu.async_copy(table.at[idx_v], rows_v, sem).wait()
            pltpu.sync_copy(rows_v, out.at[r, 0, pl.ds(c0, b_per_w)])

    return sc_gather


def kernel(x, position_ids, cos_cached, sin_cached):
    b, s = position_ids.shape
    cos4, sin4 = _build_sc_gather(b, s)(
        position_ids.astype(jnp.int32),
        cos_cached.astype(jnp.float32), sin_cached.astype(jnp.float32))
    return cos4.astype(x.dtype), sin4.astype(x.dtype)


# speedup vs baseline: 1.0393x; 1.0303x over previous
import functools

import jax
import jax.numpy as jnp
from jax import lax
from jax.experimental import pallas as pl
from jax.experimental.pallas import tpu as pltpu
from jax.experimental.pallas import tpu_sc as plsc

DIM = 128
_NC, _NS = 2, 16
_NW = _NC * _NS
_TAIL = 8


def out_slice(out, r, c0, n):
    return out.at[r, 0, pl.ds(c0, n)]


@functools.lru_cache(maxsize=None)
def _build_sc_gather(b: int, s: int):
    n_rows = b * s
    assert n_rows % (8 * _NW) == 0
    b_per_w = n_rows // _NW
    w_per_row = s // b_per_w
    mesh = plsc.VectorSubcoreMesh(core_axis_name="c", subcore_axis_name="s")

    @functools.partial(
        pl.kernel,
        mesh=mesh,
        out_type=[
            jax.ShapeDtypeStruct((b, 1, s, DIM), jnp.float32),
            jax.ShapeDtypeStruct((b, 1, s, DIM), jnp.float32),
        ],
        scratch_types=[
            pltpu.VMEM((b_per_w,), jnp.int32),
            pltpu.VMEM((b_per_w, DIM), jnp.float32),
            pltpu.VMEM((b_per_w - _TAIL, DIM), jnp.float32),
            pltpu.SemaphoreType.DMA,
            pltpu.SemaphoreType.DMA,
            pltpu.SemaphoreType.DMA,
        ],
    )
    def sc_gather(pos_hbm, cos_hbm, sin_hbm, cos_out, sin_out,
                  idx_v, cos_v, sin_v, sem_c, sem_s, sem_w):
        wid = lax.axis_index("s") * _NC + lax.axis_index("c")
        r = wid // w_per_row
        c0 = (wid % w_per_row) * b_per_w
        head = b_per_w - _TAIL
        pltpu.sync_copy(pos_hbm.at[r, pl.ds(c0, b_per_w)], idx_v)
        g_c = pltpu.async_copy(cos_hbm.at[idx_v], cos_v, sem_c)
        g_s = pltpu.async_copy(sin_hbm.at[idx_v.at[pl.ds(0, head)]],
                               sin_v, sem_s)
        g_c.wait()
        w_c = pltpu.async_copy(cos_v, out_slice(cos_out, r, c0, b_per_w),
                               sem_w)
        g_s.wait()
        w_s = pltpu.async_copy(sin_v, out_slice(sin_out, r, c0, head), sem_w)
        w_c.wait()
        g_t = pltpu.async_copy(sin_hbm.at[idx_v.at[pl.ds(head, _TAIL)]],
                               cos_v.at[pl.ds(0, _TAIL)], sem_s)
        g_t.wait()
        w_t = pltpu.async_copy(cos_v.at[pl.ds(0, _TAIL)],
                               out_slice(sin_out, r, c0 + head, _TAIL),
                               sem_w)
        w_s.wait()
        w_t.wait()

    return sc_gather


def kernel(x, position_ids, cos_cached, sin_cached):
    b, s = position_ids.shape
    cos4, sin4 = _build_sc_gather(b, s)(
        position_ids.astype(jnp.int32),
        cos_cached.astype(jnp.float32), sin_cached.astype(jnp.float32))
    return cos4.astype(x.dtype), sin4.astype(x.dtype)
